# baseline (speedup 1.0000x reference)
import jax
import jax.numpy as jnp
from jax import lax
from jax.experimental import pallas as pl
from jax.experimental.pallas import tpu as pltpu
from jax.experimental.pallas import tpu_sc as plsc

MAXLEN = 200
EMBED = 64
BATCH = 4096
NC = 2
NS = 16
NW = NC * NS
SEQ_PER_W = BATCH // NW
ROWS = BATCH * MAXLEN
VOCAB = 1000000
XPAD = 208
PARTS = ((0, 104), (104, 96))


def _body(xf, tokp, pos, out, xb0, xb1, r0, r1, pos_v,
          is0, is1, gs0, gs1, ss0, ss1):
    wid = lax.axis_index("s") * NC + lax.axis_index("c")
    chunk0 = wid * SEQ_PER_W
    pltpu.sync_copy(pos, pos_v)

    bufs = ((xb0, r0, is0, gs0, ss0), (xb1, r1, is1, gs1, ss1))

    def stage_a(g, b):
        xblk, _, isem, _, _ = bufs[b]
        gg = lax.min(g, SEQ_PER_W - 1)
        flat0 = (chunk0 + gg) * MAXLEN
        pltpu.async_copy(xf.at[pl.ds(flat0, MAXLEN)],
                         xblk.at[pl.ds(0, MAXLEN)], isem)

    def stage_b(g, b, wait_store):
        xblk, rows, isem, gsem, ssem = bufs[b]
        gg = lax.min(g, SEQ_PER_W - 1)
        flat0 = (chunk0 + gg) * MAXLEN
        if wait_store:
            pltpu.make_async_copy(
                rows.at[pl.ds(0, MAXLEN), pl.ds(0, EMBED)],
                out.at[pl.ds(flat0, MAXLEN), pl.ds(0, EMBED)],
                ssem).wait()
        pltpu.make_async_copy(xf.at[pl.ds(0, MAXLEN)],
                              xblk.at[pl.ds(0, MAXLEN)], isem).wait()
        for o, n in PARTS:
            pltpu.async_copy(
                tokp.at[xblk.at[pl.ds(o, n)]],
                rows.at[pl.ds(o, n)], gsem)

    def stage_c(g, b):
        xblk, rows, _, gsem, ssem = bufs[b]
        flat0 = (chunk0 + g) * MAXLEN
        for o, n in PARTS:
            pltpu.make_async_copy(
                tokp.at[xblk.at[pl.ds(o, n)]],
                rows.at[pl.ds(o, n)], gsem).wait()

        def addl(k, carry):
            for u in range(4):
                r = k * 4 + u
                for d in range(EMBED // 16):
                    sl = pl.ds(d * 16, 16)
                    rows[r, sl] = rows[r, sl] + pos_v[r, sl]
            return carry

        lax.fori_loop(0, MAXLEN // 4, addl, 0)
        pltpu.async_copy(rows.at[pl.ds(0, MAXLEN), pl.ds(0, EMBED)],
                         out.at[pl.ds(flat0, MAXLEN), pl.ds(0, EMBED)], ssem)

    stage_a(0, 0)
    stage_a(1, 1)
    stage_b(0, 0, False)
    stage_b(1, 1, False)

    def pair(h, carry):
        g = 2 * h
        stage_c(g, 0)
        stage_a(g + 2, 0)
        stage_b(g + 2, 0, True)
        stage_c(g + 1, 1)
        stage_a(g + 3, 1)
        stage_b(g + 3, 1, True)
        return carry

    lax.fori_loop(0, SEQ_PER_W // 2, pair, 0)

    for b in (0, 1):
        xblk, rows, _, gsem, _ = bufs[b]
        for o, n in PARTS:
            pltpu.make_async_copy(
                tokp.at[xblk.at[pl.ds(o, n)]],
                rows.at[pl.ds(o, n)], gsem).wait()


@jax.jit
def _run(xf, tokp, pos):
    mesh = plsc.VectorSubcoreMesh(core_axis_name="c", subcore_axis_name="s")
    f = pl.kernel(
        _body,
        out_type=jax.ShapeDtypeStruct((ROWS, 2 * EMBED), jnp.float32),
        mesh=mesh,
        scratch_types=[
            pltpu.VMEM((XPAD,), jnp.int32),
            pltpu.VMEM((XPAD,), jnp.int32),
            pltpu.VMEM((MAXLEN, 2 * EMBED), jnp.float32),
            pltpu.VMEM((MAXLEN, 2 * EMBED), jnp.float32),
            pltpu.VMEM((MAXLEN, EMBED), jnp.float32),
            pltpu.SemaphoreType.DMA,
            pltpu.SemaphoreType.DMA,
            pltpu.SemaphoreType.DMA,
            pltpu.SemaphoreType.DMA,
            pltpu.SemaphoreType.DMA,
            pltpu.SemaphoreType.DMA,
        ],
        compiler_params=pltpu.CompilerParams(use_tc_tiling_on_sc=False),
    )
    return f(xf, tokp, pos)


def kernel(x, token_table, pos_table):
    xf = x.astype(jnp.int32).reshape(ROWS)
    tokp = jnp.pad(token_table, ((0, 0), (0, EMBED)))
    out128 = _run(xf, tokp, pos_table)
    out = lax.slice(out128, (0, 0), (ROWS, EMBED))
    return out.reshape(BATCH, MAXLEN, EMBED)


# speedup vs baseline: 1.4018x; 1.4018x over previous
import jax
import jax.numpy as jnp
from jax import lax
from jax.experimental import pallas as pl
from jax.experimental.pallas import tpu as pltpu
from jax.experimental.pallas import tpu_sc as plsc

MAXLEN = 200
EMBED = 64
BATCH = 4096
NC = 2
NS = 16
NW = NC * NS
SEQ_PER_W = BATCH // NW
ROWS = BATCH * MAXLEN
VOCAB = 1000000
XPAD = 208
PARTS = ((0, 104), (104, 96))


def _body(xf, tokp, pos, out, xb0, xb1, r0, r1, pos_v,
          is0, is1, gs0, gs1, ss0, ss1):
    wid = lax.axis_index("s") * NC + lax.axis_index("c")
    chunk0 = wid * SEQ_PER_W
    pltpu.sync_copy(pos, pos_v)

    bufs = ((xb0, r0, is0, gs0, ss0), (xb1, r1, is1, gs1, ss1))

    def stage_a(g, b):
        xblk, _, isem, _, _ = bufs[b]
        gg = lax.min(g, SEQ_PER_W - 1)
        flat0 = (chunk0 + gg) * MAXLEN
        pltpu.async_copy(xf.at[pl.ds(flat0, MAXLEN)],
                         xblk.at[pl.ds(0, MAXLEN)], isem)

    def stage_b(g, b, wait_store):
        xblk, rows, isem, gsem, ssem = bufs[b]
        gg = lax.min(g, SEQ_PER_W - 1)
        flat0 = (chunk0 + gg) * MAXLEN
        if wait_store:
            pltpu.make_async_copy(
                rows, out.at[pl.ds(flat0, MAXLEN), pl.ds(0, EMBED)],
                ssem).wait()
        pltpu.make_async_copy(xf.at[pl.ds(0, MAXLEN)],
                              xblk.at[pl.ds(0, MAXLEN)], isem).wait()
        for o, n in PARTS:
            pltpu.async_copy(
                tokp.at[xblk.at[pl.ds(o, n)]],
                rows.at[pl.ds(o, n)], gsem)

    def stage_c(g, b):
        xblk, rows, _, gsem, ssem = bufs[b]
        flat0 = (chunk0 + g) * MAXLEN
        for o, n in PARTS:
            pltpu.make_async_copy(
                tokp.at[xblk.at[pl.ds(o, n)]],
                rows.at[pl.ds(o, n)], gsem).wait()

        def addl(k, carry):
            for u in range(4):
                r = k * 4 + u
                for d in range(EMBED // 16):
                    sl = pl.ds(d * 16, 16)
                    rows[r, sl] = rows[r, sl] + pos_v[r, sl]
            return carry

        lax.fori_loop(0, MAXLEN // 4, addl, 0)
        pltpu.async_copy(rows,
                         out.at[pl.ds(flat0, MAXLEN), pl.ds(0, EMBED)], ssem)

    stage_a(0, 0)
    stage_a(1, 1)
    stage_b(0, 0, False)
    stage_b(1, 1, False)

    def pair(h, carry):
        g = 2 * h
        stage_c(g, 0)
        stage_a(g + 2, 0)
        stage_b(g + 2, 0, True)
        stage_c(g + 1, 1)
        stage_a(g + 3, 1)
        stage_b(g + 3, 1, True)
        return carry

    lax.fori_loop(0, SEQ_PER_W // 2, pair, 0)

    for b in (0, 1):
        xblk, rows, _, gsem, _ = bufs[b]
        for o, n in PARTS:
            pltpu.make_async_copy(
                tokp.at[xblk.at[pl.ds(o, n)]],
                rows.at[pl.ds(o, n)], gsem).wait()


@jax.jit
def _run(xf, tokp, pos):
    mesh = plsc.VectorSubcoreMesh(core_axis_name="c", subcore_axis_name="s")
    f = pl.kernel(
        _body,
        out_type=jax.ShapeDtypeStruct((ROWS, 2 * EMBED), jnp.float32),
        mesh=mesh,
        scratch_types=[
            pltpu.VMEM((XPAD,), jnp.int32),
            pltpu.VMEM((XPAD,), jnp.int32),
            pltpu.VMEM((MAXLEN, EMBED), jnp.float32),
            pltpu.VMEM((MAXLEN, EMBED), jnp.float32),
            pltpu.VMEM((MAXLEN, EMBED), jnp.float32),
            pltpu.SemaphoreType.DMA,
            pltpu.SemaphoreType.DMA,
            pltpu.SemaphoreType.DMA,
            pltpu.SemaphoreType.DMA,
            pltpu.SemaphoreType.DMA,
            pltpu.SemaphoreType.DMA,
        ],
        compiler_params=pltpu.CompilerParams(use_tc_tiling_on_sc=False),
    )
    return f(xf, tokp, pos)


def kernel(x, token_table, pos_table):
    xf = x.astype(jnp.int32).reshape(ROWS)
    out128 = _run(xf, token_table, pos_table)
    out = lax.slice(out128, (0, 0), (ROWS, EMBED))
    return out.reshape(BATCH, MAXLEN, EMBED)
